# Initial kernel scaffold; baseline (speedup 1.0000x reference)
import jax
import jax.numpy as jnp
from jax.experimental import pallas as pl


def kernel(node_embeddings, edge_index):
    raise NotImplementedError("write your pallas kernel here")


# speedup vs baseline: 1.1732x; 1.1732x over previous
import functools

import jax
import jax.numpy as jnp
from jax import lax
from jax.experimental import pallas as pl
from jax.experimental.pallas import tpu as pltpu
from jax.experimental.pallas import tpu_sc as plsc

N_NODES = 10000
D_FEAT = 128
N_EDGES = 320000

NC = 2
NS = 16
NW = NC * NS
EPW = N_EDGES // NW
B = 80
NB = EPW // B
NG = B // 16


def _make_edge_dot():
  mesh = plsc.VectorSubcoreMesh(core_axis_name="c", subcore_axis_name="s")

  @functools.partial(
      pl.kernel,
      out_type=jax.ShapeDtypeStruct((N_EDGES,), jnp.float32),
      mesh=mesh,
      compiler_params=pltpu.CompilerParams(needs_layout_passes=False),
      scratch_types=[
          pltpu.VMEM((EPW,), jnp.int32),
          pltpu.VMEM((EPW,), jnp.int32),
          pltpu.VMEM((B, D_FEAT), jnp.float32),
          pltpu.VMEM((B, D_FEAT), jnp.float32),
          pltpu.VMEM((B, D_FEAT), jnp.float32),
          pltpu.VMEM((B, D_FEAT), jnp.float32),
          pltpu.VMEM((B,), jnp.float32),
          pltpu.SemaphoreType.DMA,
          pltpu.SemaphoreType.DMA,
      ],
  )
  def edge_dot(table, src_h, dst_h, out_h, isv, idv, rs0, rd0, rs1, rd1, ob,
               sem0, sem1):
    wid = lax.axis_index("s") * NC + lax.axis_index("c")
    base = wid * EPW
    pltpu.sync_copy(src_h.at[pl.ds(base, EPW)], isv)
    pltpu.sync_copy(dst_h.at[pl.ds(base, EPW)], idv)

    lanes = lax.iota(jnp.int32, 16)
    row_ids = [lanes + 16 * g for g in range(NG)]

    def gathers(i, rs, rd, sem):
      a = pltpu.make_async_copy(table.at[isv.at[pl.ds(i * B, B)]], rs, sem)
      b = pltpu.make_async_copy(table.at[idv.at[pl.ds(i * B, B)]], rd, sem)
      return a, b

    def start(i, rs, rd, sem):
      a, b = gathers(i, rs, rd, sem)
      a.start()
      b.start()

    def wait(i, rs, rd, sem):
      a, b = gathers(i, rs, rd, sem)
      a.wait()
      b.wait()

    def compute(i, rs, rd):
      def dbody(d, accs):
        col = jnp.full((16,), d, jnp.int32)
        return tuple(
            accs[g]
            + plsc.load_gather(rs, [row_ids[g], col])
            * plsc.load_gather(rd, [row_ids[g], col])
            for g in range(NG))
      accs = lax.fori_loop(
          0, D_FEAT, dbody,
          tuple(jnp.zeros((16,), jnp.float32) for _ in range(NG)),
          unroll=2)
      for g in range(NG):
        ob[pl.ds(16 * g, 16)] = accs[g]
      pltpu.sync_copy(ob, out_h.at[pl.ds(base + i * B, B)])

    start(0, rs0, rd0, sem0)
    start(1, rs1, rd1, sem1)

    def pair(k, carry):
      i0 = 2 * k
      wait(i0, rs0, rd0, sem0)
      compute(i0, rs0, rd0)
      start(i0 + 2, rs0, rd0, sem0)

      i1 = i0 + 1
      wait(i1, rs1, rd1, sem1)
      compute(i1, rs1, rd1)

      @pl.when(i1 + 2 < NB)
      def _():
        start(i1 + 2, rs1, rd1, sem1)

      return carry

    lax.fori_loop(0, (NB - 1) // 2, pair, 0)
    wait(NB - 1, rs0, rd0, sem0)
    compute(NB - 1, rs0, rd0)

  return edge_dot


@functools.lru_cache(maxsize=1)
def _edge_dot_kernel():
  return _make_edge_dot()


def kernel(node_embeddings, edge_index):
  ei = edge_index.astype(jnp.int32)
  return _edge_dot_kernel()(node_embeddings, ei[0], ei[1])


# speedup vs baseline: 1.3365x; 1.1391x over previous
import functools

import jax
import jax.numpy as jnp
from jax import lax
from jax.experimental import pallas as pl
from jax.experimental.pallas import tpu as pltpu
from jax.experimental.pallas import tpu_sc as plsc

N_NODES = 10000
D_FEAT = 128
N_EDGES = 320000

NC = 2
NS = 16
NW = NC * NS
EPW = N_EDGES // NW
B = 16
NB = EPW // B
SLOTS = 4
NSTAGE = 10
RPS = N_NODES // NSTAGE


def _make_edge_dot():
  mesh = plsc.VectorSubcoreMesh(core_axis_name="c", subcore_axis_name="s")

  @functools.partial(
      pl.kernel,
      out_type=jax.ShapeDtypeStruct((N_EDGES,), jnp.float32),
      mesh=mesh,
      compiler_params=pltpu.CompilerParams(needs_layout_passes=False),
      scratch_types=[
          pltpu.VMEM_SHARED((N_NODES, D_FEAT), jnp.float32),
          pltpu.VMEM((EPW,), jnp.int32),
          pltpu.VMEM((EPW,), jnp.int32),
          [pltpu.VMEM((B, D_FEAT), jnp.float32) for _ in range(SLOTS)],
          [pltpu.VMEM((B, D_FEAT), jnp.float32) for _ in range(SLOTS)],
          [pltpu.VMEM((B,), jnp.float32) for _ in range(SLOTS)],
          [pltpu.SemaphoreType.DMA for _ in range(SLOTS)],
          [pltpu.SemaphoreType.DMA for _ in range(SLOTS)],
      ],
  )
  def edge_dot(table, src_h, dst_h, out_h, shtab, isv, idv, rs, rd, ob,
               semg, semo):
    sid = lax.axis_index("s")
    wid = sid * NC + lax.axis_index("c")
    base = wid * EPW

    @pl.when(sid < NSTAGE)
    def _():
      pltpu.sync_copy(table.at[pl.ds(sid * RPS, RPS)],
                      shtab.at[pl.ds(sid * RPS, RPS)])
    pltpu.sync_copy(src_h.at[pl.ds(base, EPW)], isv)
    pltpu.sync_copy(dst_h.at[pl.ds(base, EPW)], idv)
    plsc.subcore_barrier()

    lanes = lax.iota(jnp.int32, 16)

    def gathers(i, b):
      a = pltpu.make_async_copy(shtab.at[isv.at[pl.ds(i * B, B)]], rs[b],
                                semg[b])
      c = pltpu.make_async_copy(shtab.at[idv.at[pl.ds(i * B, B)]], rd[b],
                                semg[b])
      return a, c

    def start(i, b):
      a, c = gathers(i, b)
      a.start()
      c.start()

    def wait(i, b):
      a, c = gathers(i, b)
      a.wait()
      c.wait()

    def out_copy(i, b):
      return pltpu.make_async_copy(ob[b], out_h.at[pl.ds(base + i * B, B)],
                                   semo[b])

    def compute(i, b):
      def dbody(d, acc):
        col = jnp.full((16,), d, jnp.int32)
        return (acc
                + plsc.load_gather(rs[b], [lanes, col])
                * plsc.load_gather(rd[b], [lanes, col]))
      acc = lax.fori_loop(0, D_FEAT, dbody, jnp.zeros((16,), jnp.float32),
                          unroll=8)

      @pl.when(i >= SLOTS)
      def _():
        out_copy(i - SLOTS, b).wait()

      ob[b][...] = acc
      out_copy(i, b).start()

    for b in range(SLOTS):
      start(b, b)

    def quad(k, carry):
      i0 = SLOTS * k
      for b in range(SLOTS):
        i = i0 + b
        wait(i, b)
        compute(i, b)
        nxt = i + SLOTS

        @pl.when(nxt < NB)
        def _():
          start(nxt, b)
      return carry

    lax.fori_loop(0, NB // SLOTS, quad, 0)
    wait(NB - 1, 0)
    compute(NB - 1, 0)
    for b, i in ((1, NB - 4), (2, NB - 3), (3, NB - 2), (0, NB - 1)):
      out_copy(i, b).wait()

  return edge_dot


@functools.lru_cache(maxsize=1)
def _edge_dot_kernel():
  return _make_edge_dot()


def kernel(node_embeddings, edge_index):
  ei = edge_index.astype(jnp.int32)
  return _edge_dot_kernel()(node_embeddings, ei[0], ei[1])


# speedup vs baseline: 2.4543x; 1.8364x over previous
import functools

import jax
import jax.numpy as jnp
from jax import lax
from jax.experimental import pallas as pl
from jax.experimental.pallas import tpu as pltpu
from jax.experimental.pallas import tpu_sc as plsc

N_NODES = 10000
D_FEAT = 128
DW = D_FEAT // 2
N_EDGES = 320000

NC = 2
NS = 16
NW = NC * NS
EPW = N_EDGES // NW
B = 16
NB = EPW // B
SLOTS = 4
NSTAGE = 10
RPS = N_NODES // NSTAGE


def _make_edge_dot():
  mesh = plsc.VectorSubcoreMesh(core_axis_name="c", subcore_axis_name="s")

  @functools.partial(
      pl.kernel,
      out_type=jax.ShapeDtypeStruct((N_EDGES,), jnp.float32),
      mesh=mesh,
      compiler_params=pltpu.CompilerParams(
          needs_layout_passes=False, use_tc_tiling_on_sc=False),
      scratch_types=[
          pltpu.VMEM_SHARED((N_NODES, DW), jnp.int32),
          pltpu.VMEM((EPW,), jnp.int32),
          pltpu.VMEM((EPW,), jnp.int32),
          [pltpu.VMEM((B, DW), jnp.int32) for _ in range(SLOTS)],
          [pltpu.VMEM((B, DW), jnp.int32) for _ in range(SLOTS)],
          [pltpu.VMEM((B,), jnp.float32) for _ in range(SLOTS)],
          [pltpu.SemaphoreType.DMA for _ in range(SLOTS)],
          [pltpu.SemaphoreType.DMA for _ in range(SLOTS)],
      ],
  )
  def edge_dot(table, src_h, dst_h, out_h, shtab, isv, idv, rs, rd, ob,
               semg, semo):
    sid = lax.axis_index("s")
    wid = sid * NC + lax.axis_index("c")
    base = wid * EPW

    @pl.when(sid < NSTAGE)
    def _():
      pltpu.sync_copy(table.at[pl.ds(sid * RPS, RPS)],
                      shtab.at[pl.ds(sid * RPS, RPS)])
    pltpu.sync_copy(src_h.at[pl.ds(base, EPW)], isv)
    pltpu.sync_copy(dst_h.at[pl.ds(base, EPW)], idv)
    plsc.subcore_barrier()

    lanes = lax.iota(jnp.int32, 16)

    def gathers(i, b):
      a = pltpu.make_async_copy(shtab.at[isv.at[pl.ds(i * B, B)]], rs[b],
                                semg[b])
      c = pltpu.make_async_copy(shtab.at[idv.at[pl.ds(i * B, B)]], rd[b],
                                semg[b])
      return a, c

    def start(i, b):
      a, c = gathers(i, b)
      a.start()
      c.start()

    def wait(i, b):
      a, c = gathers(i, b)
      a.wait()
      c.wait()

    def out_copy(i, b):
      return pltpu.make_async_copy(ob[b], out_h.at[pl.ds(base + i * B, B)],
                                   semo[b])

    def compute(i, b):
      def dbody(d, acc):
        col = jnp.full((16,), d, jnp.int32)
        pa = plsc.load_gather(rs[b], [lanes, col])
        pb = plsc.load_gather(rd[b], [lanes, col])
        a0, a1 = plsc.unpack(plsc.bitcast(pa, jnp.bfloat16),
                             format=plsc.PackFormat.INTERLEAVED)
        b0, b1 = plsc.unpack(plsc.bitcast(pb, jnp.bfloat16),
                             format=plsc.PackFormat.INTERLEAVED)
        return acc + a0 * b0 + a1 * b1
      acc = lax.fori_loop(0, DW, dbody, jnp.zeros((16,), jnp.float32),
                          unroll=8)

      @pl.when(i >= SLOTS)
      def _():
        out_copy(i - SLOTS, b).wait()

      ob[b][...] = acc
      out_copy(i, b).start()

    for b in range(SLOTS):
      start(b, b)

    def quad(k, carry):
      i0 = SLOTS * k
      for b in range(SLOTS):
        i = i0 + b
        wait(i, b)
        compute(i, b)
        nxt = i + SLOTS

        @pl.when(nxt < NB)
        def _():
          start(nxt, b)
      return carry

    lax.fori_loop(0, NB // SLOTS, quad, 0)
    wait(NB - 1, 0)
    compute(NB - 1, 0)
    for b, i in ((1, NB - 4), (2, NB - 3), (3, NB - 2), (0, NB - 1)):
      out_copy(i, b).wait()

  return edge_dot


@functools.lru_cache(maxsize=1)
def _edge_dot_kernel():
  return _make_edge_dot()


def kernel(node_embeddings, edge_index):
  ei = edge_index.astype(jnp.int32)
  packed = jax.lax.bitcast_convert_type(
      node_embeddings.astype(jnp.bfloat16).reshape(N_NODES, DW, 2), jnp.int32)
  return _edge_dot_kernel()(packed, ei[0], ei[1])
